# baseline (speedup 1.0000x reference)
import functools

import jax
import jax.numpy as jnp
from jax import lax
from jax.experimental import pallas as pl
from jax.experimental.pallas import tpu as pltpu
from jax.experimental.pallas import tpu_sc as plsc

VOCAB = 1000000
EMBED_DIM = 32
BATCH = 4096
HIST = 200

NC = 2
NS = 16
NW = NC * NS

LANE = 128
NBLOCK = VOCAB // LANE
TAIL_START = NBLOCK * LANE
TAIL_ROWS = (VOCAB - TAIL_START) * EMBED_DIM // LANE
TLIN_ROWS = VOCAB * EMBED_DIM // LANE
BLOCKS_EVEN = (NBLOCK // NW) & ~3
AQUADS = BLOCKS_EVEN // 4
ALEFT = NBLOCK - BLOCKS_EVEN * NW

ROWS_W = BATCH // NW
BQUADS = HIST // 4

_MESH = plsc.VectorSubcoreMesh(
    core_axis_name="c", subcore_axis_name="s", num_cores=NC, num_subcores=NS
)


def _iota16():
  return lax.iota(jnp.int32, 16)


def _make_transpose_kernel():
  @functools.partial(
      pl.kernel,
      out_type=jax.ShapeDtypeStruct((TLIN_ROWS * LANE,), jnp.float32),
      mesh=_MESH,
      scratch_types=[
          pltpu.VMEM((4, EMBED_DIM, LANE), jnp.float32),
          pltpu.VMEM((EMBED_DIM * LANE,), jnp.float32),
          pltpu.VMEM((EMBED_DIM * LANE,), jnp.float32),
          pltpu.VMEM((EMBED_DIM * LANE,), jnp.float32),
          pltpu.VMEM((EMBED_DIM * LANE,), jnp.float32),
          pltpu.SemaphoreType.DMA((4,)),
          pltpu.SemaphoreType.DMA((4,)),
      ],
      compiler_params=pltpu.CompilerParams(use_tc_tiling_on_sc=True, needs_layout_passes=False),
  )
  def tkernel(tabt_hbm, tail_hbm, tlin_hbm, va, vt0, vt1, vt2, vt3,
              sem_in, sem_out):
    vt = (vt0, vt1, vt2, vt3)
    wid = lax.axis_index("s") * NC + lax.axis_index("c")
    iota = _iota16()

    def fire_in(s, c):
      col0 = pl.multiple_of(c * LANE, LANE)
      pltpu.async_copy(
          tabt_hbm.at[:, pl.ds(col0, LANE)], va.at[s], sem_in.at[s]
      )

    def wait_in(s):
      pltpu.make_async_copy(
          tabt_hbm.at[:, pl.ds(0, LANE)], va.at[s], sem_in.at[s]
      ).wait()

    def fire_out(s, c):
      el0 = pl.multiple_of(c * EMBED_DIM * LANE, 8)
      pltpu.async_copy(
          vt[s], tlin_hbm.at[pl.ds(el0, EMBED_DIM * LANE)], sem_out.at[s]
      )

    def wait_out(s):
      pltpu.make_async_copy(
          vt[s], tlin_hbm.at[pl.ds(0, EMBED_DIM * LANE)], sem_out.at[s]
      ).wait()

    perms = [lax.rem(iota + j, 16) for j in range(16)]

    def transpose(s):
      src = va.at[s]
      dst = vt[s]
      for dh in range(EMBED_DIM // 16):
        for j in range(16):
          ddiag = perms[j] + 16 * dh
          flatbase = iota * EMBED_DIM + ddiag

          @plsc.parallel_loop(0, 8, unroll=2)
          def _(q):
            v0 = q * 16
            colv = iota + v0
            x = plsc.load_gather(src, [ddiag, colv])
            plsc.store_scatter(dst, [flatbase + v0 * EMBED_DIM], x)

    for s in range(4):
      fire_in(s, wid + s * NW)

    def body(p, carry):
      for s in range(4):
        c = wid + NW * (4 * p + s)
        wait_in(s)
        pl.when(p >= 1)(lambda s=s: wait_out(s))
        transpose(s)
        fire_out(s, c)
        pl.when(p < AQUADS - 1)(lambda s=s, c=c: fire_in(s, c + 4 * NW))
      return carry

    lax.fori_loop(0, AQUADS, body, 0)
    for s in range(4):
      wait_out(s)

    @pl.when(wid < ALEFT)
    def _():
      c = NW * BLOCKS_EVEN + wid
      col0 = pl.multiple_of(c * LANE, LANE)
      pltpu.sync_copy(tabt_hbm.at[:, pl.ds(col0, LANE)], va.at[0])
      transpose(0)
      el0 = pl.multiple_of(c * EMBED_DIM * LANE, 8)
      pltpu.sync_copy(vt[0], tlin_hbm.at[pl.ds(el0, EMBED_DIM * LANE)])

    @pl.when(wid == ALEFT)
    def _():
      n = TAIL_ROWS * LANE
      dst = vt[0].at[pl.ds(0, n)]
      pltpu.sync_copy(tail_hbm, dst)
      pltpu.sync_copy(dst, tlin_hbm.at[pl.ds(TLIN_ROWS * LANE - n, n)])

  return tkernel


def _make_gather_kernel():
  @functools.partial(
      pl.kernel,
      out_type=jax.ShapeDtypeStruct(
          (HIST, EMBED_DIM // 8, BATCH // LANE, 8, LANE), jnp.float32
      ),
      mesh=_MESH,
      scratch_types=[
          pltpu.VMEM((HIST, ROWS_W), jnp.int32),
          pltpu.VMEM((4, ROWS_W, EMBED_DIM), jnp.float32),
          pltpu.VMEM((4, EMBED_DIM // 8, 8, LANE), jnp.float32),
          pltpu.SemaphoreType.DMA((4,)),
          pltpu.SemaphoreType.DMA((4,)),
      ],
      compiler_params=pltpu.CompilerParams(use_tc_tiling_on_sc=False, needs_layout_passes=False),
  )
  def gkernel(tlin_hbm, idxt_hbm, out_hbm, idx_v, rows, vtile, sem_g, sem_o):
    wid = lax.axis_index("s") * NC + lax.axis_index("c")
    b0 = wid * ROWS_W
    iota = _iota16()
    pltpu.sync_copy(idxt_hbm.at[:, pl.ds(b0, ROWS_W)], idx_v)

    def fire_gather(s, h):
      pltpu.async_copy(tlin_hbm.at[idx_v.at[h]], rows.at[s], sem_g.at[s])

    def wait_gather(s):
      pltpu.make_async_copy(
          tlin_hbm.at[idx_v.at[0]], rows.at[s], sem_g.at[s]
      ).wait()

    def fire_out(s, h):
      pltpu.async_copy(vtile.at[s], out_hbm.at[h, :, wid], sem_o.at[s])

    def wait_out(s):
      pltpu.make_async_copy(
          vtile.at[s], out_hbm.at[0, :, wid], sem_o.at[s]
      ).wait()

    perms = [lax.rem(iota + j, 16) for j in range(16)]

    def transpose(s):
      src = rows.at[s]
      dst = vtile.at[s]
      for dh in range(EMBED_DIM // 16):
        for j in range(16):
          ddiag = perms[j] + 16 * dh
          trv = ddiag // 8
          s8v = lax.rem(ddiag, 8)

          @plsc.parallel_loop(0, ROWS_W // 16, unroll=2)
          def _(q):
            lvec = iota + q * 16
            x = plsc.load_gather(src, [lvec, ddiag])
            plsc.store_scatter(dst, [trv, s8v, lvec], x)

    for s in range(4):
      fire_gather(s, s)

    def body(p, carry):
      for s in range(4):
        h = 4 * p + s
        wait_gather(s)
        pl.when(p >= 1)(lambda s=s: wait_out(s))
        transpose(s)
        fire_out(s, h)
        pl.when(p < BQUADS - 1)(lambda s=s, h=h: fire_gather(s, h + 4))
      return carry

    lax.fori_loop(0, BQUADS, body, 0)
    for s in range(4):
      wait_out(s)

  return gkernel


_transpose_table = _make_transpose_kernel()
_gather = _make_gather_kernel()


@jax.jit
def kernel(indices, table):
  idxt = indices.astype(jnp.int32).T
  tail = table[TAIL_START:, :].reshape(TAIL_ROWS * LANE)
  tlin = _transpose_table(table.T, tail)
  out5 = _gather(tlin.reshape(VOCAB, EMBED_DIM), idxt)
  return out5.transpose(2, 4, 0, 1, 3).reshape(BATCH, HIST, EMBED_DIM)


# speedup vs baseline: 1.9789x; 1.9789x over previous
import functools

import jax
import jax.numpy as jnp
from jax import lax
from jax.experimental import pallas as pl
from jax.experimental.pallas import tpu as pltpu
from jax.experimental.pallas import tpu_sc as plsc

VOCAB = 1000000
EMBED_DIM = 32
BATCH = 4096
HIST = 200

NC = 2
NS = 16
NW = NC * NS

LANE = 128
NBLOCK = VOCAB // LANE
TAIL_START = NBLOCK * LANE
TAIL_ROWS = (VOCAB - TAIL_START) * EMBED_DIM // LANE
TLIN_ROWS = VOCAB * EMBED_DIM // LANE
BLOCKS_EVEN = (NBLOCK // NW) & ~3
AQUADS = BLOCKS_EVEN // 4
ALEFT = NBLOCK - BLOCKS_EVEN * NW

ROWS_W = BATCH // NW
BQUADS = HIST // 4

_MESH = plsc.VectorSubcoreMesh(
    core_axis_name="c", subcore_axis_name="s", num_cores=NC, num_subcores=NS
)


def _iota16():
  return lax.iota(jnp.int32, 16)


def _make_transpose_kernel():
  @functools.partial(
      pl.kernel,
      out_type=jax.ShapeDtypeStruct((TLIN_ROWS * LANE,), jnp.float32),
      mesh=_MESH,
      scratch_types=[
          pltpu.VMEM((4, EMBED_DIM, LANE), jnp.float32),
          pltpu.VMEM((EMBED_DIM * LANE,), jnp.float32),
          pltpu.VMEM((EMBED_DIM * LANE,), jnp.float32),
          pltpu.VMEM((EMBED_DIM * LANE,), jnp.float32),
          pltpu.VMEM((EMBED_DIM * LANE,), jnp.float32),
          pltpu.SemaphoreType.DMA((4,)),
          pltpu.SemaphoreType.DMA((4,)),
      ],
      compiler_params=pltpu.CompilerParams(use_tc_tiling_on_sc=True, needs_layout_passes=False),
  )
  def tkernel(tabt_hbm, tail_hbm, tlin_hbm, va, vt0, vt1, vt2, vt3,
              sem_in, sem_out):
    vt = (vt0, vt1, vt2, vt3)
    wid = lax.axis_index("s") * NC + lax.axis_index("c")
    iota = _iota16()

    def fire_in(s, c):
      col0 = pl.multiple_of(c * LANE, LANE)
      pltpu.async_copy(
          tabt_hbm.at[:, pl.ds(col0, LANE)], va.at[s], sem_in.at[s]
      )

    def wait_in(s):
      pltpu.make_async_copy(
          tabt_hbm.at[:, pl.ds(0, LANE)], va.at[s], sem_in.at[s]
      ).wait()

    def fire_out(s, c):
      el0 = pl.multiple_of(c * EMBED_DIM * LANE, 8)
      pltpu.async_copy(
          vt[s], tlin_hbm.at[pl.ds(el0, EMBED_DIM * LANE)], sem_out.at[s]
      )

    def wait_out(s):
      pltpu.make_async_copy(
          vt[s], tlin_hbm.at[pl.ds(0, EMBED_DIM * LANE)], sem_out.at[s]
      ).wait()

    perms = [lax.rem(iota + j, 16) for j in range(16)]

    def transpose(s):
      src = va.at[s]
      dst = vt[s]
      for dh in range(EMBED_DIM // 16):
        for j in range(16):
          ddiag = perms[j] + 16 * dh
          flatbase = iota * EMBED_DIM + ddiag

          @plsc.parallel_loop(0, 8, unroll=4)
          def _(q):
            v0 = q * 16
            colv = iota + v0
            x = plsc.load_gather(src, [ddiag, colv])
            plsc.store_scatter(dst, [flatbase + v0 * EMBED_DIM], x)

    for s in range(4):
      fire_in(s, wid + s * NW)

    def body(p, carry):
      for s in range(4):
        c = wid + NW * (4 * p + s)
        wait_in(s)
        pl.when(p >= 1)(lambda s=s: wait_out(s))
        transpose(s)
        fire_out(s, c)
        pl.when(p < AQUADS - 1)(lambda s=s, c=c: fire_in(s, c + 4 * NW))
      return carry

    lax.fori_loop(0, AQUADS, body, 0)
    for s in range(4):
      wait_out(s)

    @pl.when(wid < ALEFT)
    def _():
      c = NW * BLOCKS_EVEN + wid
      col0 = pl.multiple_of(c * LANE, LANE)
      pltpu.sync_copy(tabt_hbm.at[:, pl.ds(col0, LANE)], va.at[0])
      transpose(0)
      el0 = pl.multiple_of(c * EMBED_DIM * LANE, 8)
      pltpu.sync_copy(vt[0], tlin_hbm.at[pl.ds(el0, EMBED_DIM * LANE)])

    @pl.when(wid == ALEFT)
    def _():
      n = TAIL_ROWS * LANE
      dst = vt[0].at[pl.ds(0, n)]
      pltpu.sync_copy(tail_hbm, dst)
      pltpu.sync_copy(dst, tlin_hbm.at[pl.ds(TLIN_ROWS * LANE - n, n)])

  return tkernel


def _make_gather_kernel():
  @functools.partial(
      pl.kernel,
      out_type=jax.ShapeDtypeStruct(
          (HIST, EMBED_DIM // 8, BATCH // LANE, 8, LANE), jnp.float32
      ),
      mesh=_MESH,
      scratch_types=[
          pltpu.VMEM((HIST, ROWS_W), jnp.int32),
          pltpu.VMEM((4, ROWS_W, EMBED_DIM), jnp.float32),
          pltpu.VMEM((4, EMBED_DIM // 8, 8, LANE), jnp.float32),
          pltpu.SemaphoreType.DMA((4,)),
          pltpu.SemaphoreType.DMA((4,)),
      ],
      compiler_params=pltpu.CompilerParams(use_tc_tiling_on_sc=False, needs_layout_passes=False),
  )
  def gkernel(tlin_hbm, idxt_hbm, out_hbm, idx_v, rows, vtile, sem_g, sem_o):
    wid = lax.axis_index("s") * NC + lax.axis_index("c")
    b0 = wid * ROWS_W
    iota = _iota16()
    pltpu.sync_copy(idxt_hbm.at[:, pl.ds(b0, ROWS_W)], idx_v)

    def fire_gather(s, h):
      pltpu.async_copy(tlin_hbm.at[idx_v.at[h]], rows.at[s], sem_g.at[s])

    def wait_gather(s):
      pltpu.make_async_copy(
          tlin_hbm.at[idx_v.at[0]], rows.at[s], sem_g.at[s]
      ).wait()

    def fire_out(s, h):
      pltpu.async_copy(vtile.at[s], out_hbm.at[h, :, wid], sem_o.at[s])

    def wait_out(s):
      pltpu.make_async_copy(
          vtile.at[s], out_hbm.at[0, :, wid], sem_o.at[s]
      ).wait()

    perms = [lax.rem(iota + j, 16) for j in range(16)]

    def transpose(s):
      src = rows.at[s]
      dst = vtile.at[s]
      for dh in range(EMBED_DIM // 16):
        for j in range(16):
          ddiag = perms[j] + 16 * dh
          trv = ddiag // 8
          s8v = lax.rem(ddiag, 8)

          @plsc.parallel_loop(0, ROWS_W // 16, unroll=4)
          def _(q):
            lvec = iota + q * 16
            x = plsc.load_gather(src, [lvec, ddiag])
            plsc.store_scatter(dst, [trv, s8v, lvec], x)

    for s in range(4):
      fire_gather(s, s)

    def body(p, carry):
      for s in range(4):
        h = 4 * p + s
        wait_gather(s)
        pl.when(p >= 1)(lambda s=s: wait_out(s))
        transpose(s)
        fire_out(s, h)
        pl.when(p < BQUADS - 1)(lambda s=s, h=h: fire_gather(s, h + 4))
      return carry

    lax.fori_loop(0, BQUADS, body, 0)
    for s in range(4):
      wait_out(s)

  return gkernel


_transpose_table = _make_transpose_kernel()
_gather = _make_gather_kernel()


@jax.jit
def kernel(indices, table):
  idxt = indices.astype(jnp.int32).T
  tail = table[TAIL_START:, :].reshape(TAIL_ROWS * LANE)
  tlin = _transpose_table(table.T, tail)
  out5 = _gather(tlin.reshape(VOCAB, EMBED_DIM), idxt)
  return out5.transpose(2, 4, 0, 1, 3).reshape(BATCH, HIST, EMBED_DIM)
